# baseline (speedup 1.0000x reference)
import functools

import jax
import jax.numpy as jnp
from jax import lax
from jax.experimental import pallas as pl
from jax.experimental.pallas import tpu as pltpu
from jax.experimental.pallas import tpu_sc as plsc

N = 10000
E = 320000
G = 64
D_IN = 128
LAT = 32
HID = 512

NC = 2
NS = 16
NW = NC * NS
EPW = E // NW
NB = 80
EPWP = NB * 128
DROWS = 640
ACC_R = 10112
RPT = ACC_R // NS

_mesh = plsc.VectorSubcoreMesh(core_axis_name="c", subcore_axis_name="s")
_sc_params = pltpu.CompilerParams(needs_layout_passes=False,
                                  use_tc_tiling_on_sc=False)


@functools.partial(
    pl.kernel,
    out_type=jax.ShapeDtypeStruct((NW * DROWS * 16,), jnp.float32),
    mesh=_mesh,
    scratch_types=[
        pltpu.VMEM((EPWP,), jnp.int32),
        pltpu.VMEM((DROWS * 16,), jnp.float32),
    ],
    compiler_params=_sc_params,
)
def _sc_degree(dst_hbm, out_hbm, dstv, degloc):
    c = lax.axis_index("c")
    s = lax.axis_index("s")
    w = c * NS + s

    zero16 = jnp.zeros((16,), jnp.float32)
    one16 = jnp.ones((16,), jnp.float32)

    def _zero(r, _):
        degloc[pl.ds(r * 16, 16)] = zero16
        return 0

    lax.fori_loop(0, DROWS, _zero, 0)

    pltpu.sync_copy(dst_hbm.at[pl.ds(w * EPWP, EPWP)], dstv)

    def _hist(i, _):
        idxv = dstv[pl.ds(i * 16, 16)]
        plsc.addupdate_scatter(degloc, [idxv], one16)
        return 0

    lax.fori_loop(0, EPWP // 16, _hist, 0)

    pltpu.sync_copy(degloc, out_hbm.at[pl.ds(w * DROWS * 16, DROWS * 16)])


@functools.partial(
    pl.kernel,
    out_type=jax.ShapeDtypeStruct((NC, ACC_R, LAT), jnp.float32),
    mesh=_mesh,
    scratch_types=[
        pltpu.VMEM((NB, 128), jnp.int32),
        pltpu.VMEM((NB, 128), jnp.int32),
        pltpu.VMEM((128, LAT), jnp.float32),
        pltpu.VMEM((128, LAT), jnp.float32),
        pltpu.VMEM((RPT, LAT), jnp.float32),
        pltpu.VMEM_SHARED((ACC_R, LAT), jnp.float32),
        pltpu.SemaphoreType.DMA,
        pltpu.SemaphoreType.DMA,
    ],
    compiler_params=_sc_params,
)
def _sc_scatter(hp_hbm, src_hbm, dst_hbm, out_hbm, srcv, dstv, buf0, buf1, zb,
                acc, sem0, sem1):
    c = lax.axis_index("c")
    s = lax.axis_index("s")
    w = c * NS + s

    zero16 = jnp.zeros((16,), jnp.float32)

    def _zero(r, _):
        zb[r, pl.ds(0, 16)] = zero16
        zb[r, pl.ds(16, 16)] = zero16
        return 0

    lax.fori_loop(0, RPT, _zero, 0)
    pltpu.sync_copy(zb, acc.at[pl.ds(s * RPT, RPT)])
    plsc.subcore_barrier()

    pltpu.sync_copy(src_hbm.at[w], srcv)
    pltpu.sync_copy(dst_hbm.at[w], dstv)

    pltpu.async_copy(hp_hbm.at[srcv.at[0]], buf0, sem0)

    def _edges(jj, _):
        j0 = 2 * jj
        pltpu.async_copy(hp_hbm.at[srcv.at[j0 + 1]], buf1, sem1)
        pltpu.make_async_copy(hp_hbm.at[srcv.at[j0]], buf0, sem0).wait()
        pltpu.sync_copy(buf0, acc.at[dstv.at[j0]], add=True)

        @pl.when(jj < NB // 2 - 1)
        def _():
            pltpu.async_copy(hp_hbm.at[srcv.at[j0 + 2]], buf0, sem0)

        pltpu.make_async_copy(hp_hbm.at[srcv.at[j0 + 1]], buf1, sem1).wait()
        pltpu.sync_copy(buf1, acc.at[dstv.at[j0 + 1]], add=True)
        return 0

    lax.fori_loop(0, NB // 2, _edges, 0)
    plsc.subcore_barrier()

    pltpu.sync_copy(acc.at[pl.ds(s * RPT, RPT)], out_hbm.at[c, pl.ds(s * RPT, RPT)])


_P = jax.lax.Precision.DEFAULT


def _tc_prep_body(degt_ref, x_ref, w1_ref, dinv_ref, hp1_ref):
    deg = jnp.sum(degt_ref[pl.ds(0, N), :], axis=1, keepdims=True) + 1.0
    dinv = lax.rsqrt(deg)
    dinv_ref[...] = dinv
    hw = lax.dot_general(x_ref[...], w1_ref[...], (((1,), (0,)), ((), ())),
                         precision=_P)
    hp1_ref[...] = hw * dinv


def _tc_prep(degt, x, w1):
    return pl.pallas_call(
        _tc_prep_body,
        out_shape=(
            jax.ShapeDtypeStruct((N, 1), jnp.float32),
            jax.ShapeDtypeStruct((N, LAT), jnp.float32),
        ),
    )(degt, x, w1)


def _tc_layer_body(agg_ref, hpp_ref, dinv_ref, w_ref, b_ref, h_ref, hpn_ref):
    agg = agg_ref[0, pl.ds(0, N), :] + agg_ref[1, pl.ds(0, N), :]
    dinv = dinv_ref[...]
    h = jnp.tanh(dinv * (agg + hpp_ref[...]) + b_ref[...])
    h_ref[...] = h
    hpn_ref[...] = dinv * lax.dot_general(h, w_ref[...], (((1,), (0,)), ((), ())),
                                          precision=_P)


def _tc_layer(aggparts, hpp, dinv, w_next, b_prev):
    return pl.pallas_call(
        _tc_layer_body,
        out_shape=(
            jax.ShapeDtypeStruct((N, LAT), jnp.float32),
            jax.ShapeDtypeStruct((N, LAT), jnp.float32),
        ),
    )(aggparts, hpp, dinv, w_next, b_prev)


def _tc_head_body(agg_ref, hp3_ref, dinv_ref, b3_ref, h1_ref, h2_ref, idx_ref,
                  y_ref, l1w_ref, l1b_ref, l2w_ref, l2b_ref,
                  logits_ref, loss_ref, acc_ref, feat_ref, h3_ref, gf_ref):
    agg = agg_ref[0, pl.ds(0, N), :] + agg_ref[1, pl.ds(0, N), :]
    h3_ref[...] = jnp.tanh(dinv_ref[...] * (agg + hp3_ref[...]) + b3_ref[...])

    for g in range(G):
        ig = idx_ref[g]
        gf_ref[pl.ds(g, 1), pl.ds(0, LAT)] = h1_ref[pl.ds(ig, 1), :]
        gf_ref[pl.ds(g, 1), pl.ds(LAT, LAT)] = h2_ref[pl.ds(ig, 1), :]
        gf_ref[pl.ds(g, 1), pl.ds(2 * LAT, LAT)] = h3_ref[pl.ds(ig, 1), :]

    hidden = lax.dot_general(gf_ref[...], l1w_ref[...], (((1,), (0,)), ((), ())),
                             precision=_P) + l1b_ref[...]
    feat_ref[...] = hidden
    hr = jnp.maximum(hidden, 0.0)
    z = lax.dot_general(hr, l2w_ref[...], (((1,), (0,)), ((), ())),
                        precision=_P) + l2b_ref[...]
    z0 = z[:, 0:1]
    z1 = z[:, 1:2]
    m = jnp.maximum(z0, z1)
    lse = m + jnp.log(jnp.exp(z0 - m) + jnp.exp(z1 - m))
    logits_ref[...] = z - lse

    ytrue = y_ref[...] == 1
    sel = jnp.where(ytrue, z1, z0) - lse
    loss_ref[...] = -jnp.sum(sel, axis=0, keepdims=True) / G
    pred1 = z1 > z0
    eq = (pred1 == ytrue).astype(jnp.float32)
    acc_ref[...] = jnp.sum(eq, axis=0, keepdims=True) / G


def _tc_head(aggparts, hp3, dinv, b3, h1, h2, idx, y2, l1w, l1b, l2w, l2b):
    return pl.pallas_call(
        _tc_head_body,
        in_specs=[
            pl.BlockSpec(memory_space=pltpu.VMEM),
            pl.BlockSpec(memory_space=pltpu.VMEM),
            pl.BlockSpec(memory_space=pltpu.VMEM),
            pl.BlockSpec(memory_space=pltpu.VMEM),
            pl.BlockSpec(memory_space=pltpu.VMEM),
            pl.BlockSpec(memory_space=pltpu.VMEM),
            pl.BlockSpec(memory_space=pltpu.SMEM),
            pl.BlockSpec(memory_space=pltpu.VMEM),
            pl.BlockSpec(memory_space=pltpu.VMEM),
            pl.BlockSpec(memory_space=pltpu.VMEM),
            pl.BlockSpec(memory_space=pltpu.VMEM),
            pl.BlockSpec(memory_space=pltpu.VMEM),
        ],
        out_shape=(
            jax.ShapeDtypeStruct((G, 2), jnp.float32),
            jax.ShapeDtypeStruct((1, 1), jnp.float32),
            jax.ShapeDtypeStruct((1, 1), jnp.float32),
            jax.ShapeDtypeStruct((G, HID), jnp.float32),
        ),
        scratch_shapes=[
            pltpu.VMEM((N, LAT), jnp.float32),
            pltpu.VMEM((G, 3 * LAT), jnp.float32),
        ],
    )(aggparts, hp3, dinv, b3, h1, h2, idx, y2, l1w, l1b, l2w, l2b)


def kernel(x, edge_index, batch, y, W1, b1, W2, b2, W3, b3, L1w, L1b, L2w, L2b):
    src = edge_index[0]
    dst = edge_index[1]

    pad = jnp.zeros((NW, EPWP - EPW), jnp.int32)
    srcp = jnp.concatenate([src.reshape(NW, EPW), pad], axis=1)
    srcp = srcp.reshape(NW, NB, 128)
    dstp = jnp.concatenate([dst.reshape(NW, EPW), pad + N], axis=1)
    dst2 = dstp.reshape(NW * EPWP)
    dstp = dstp.reshape(NW, NB, 128)

    mask = jnp.concatenate([jnp.ones((1,), jnp.bool_), batch[1:] != batch[:-1]])
    idx = jnp.nonzero(mask, size=G, fill_value=0)[0].astype(jnp.int32)

    degparts = _sc_degree(dst2)
    degt = jnp.transpose(degparts.reshape(NW, DROWS * 16), (1, 0))

    dinv, hp1 = _tc_prep(degt, x, W1)

    agg1 = _sc_scatter(hp1, srcp, dstp)
    h1, hp2 = _tc_layer(agg1, hp1, dinv, W2, b1.reshape(1, LAT))
    agg2 = _sc_scatter(hp2, srcp, dstp)
    h2, hp3 = _tc_layer(agg2, hp2, dinv, W3, b2.reshape(1, LAT))
    agg3 = _sc_scatter(hp3, srcp, dstp)

    logits, loss, acc, feature = _tc_head(
        agg3, hp3, dinv, b3.reshape(1, LAT), h1, h2, idx,
        y.reshape(G, 1).astype(jnp.int32), L1w, L1b.reshape(1, HID),
        L2w, L2b.reshape(1, 2))

    return (logits, loss.reshape(()), acc.reshape(()), feature)


# speedup vs baseline: 1.0996x; 1.0996x over previous
import functools

import jax
import jax.numpy as jnp
from jax import lax
from jax.experimental import pallas as pl
from jax.experimental.pallas import tpu as pltpu
from jax.experimental.pallas import tpu_sc as plsc

N = 10000
E = 320000
G = 64
D_IN = 128
LAT = 32
HID = 512

NC = 2
NS = 16
NW = NC * NS
EPW = E // NW
NB = 80
EPWP = NB * 128
DROWS = 640
ACC_R = 10112
RPT = ACC_R // NS

_mesh = plsc.VectorSubcoreMesh(core_axis_name="c", subcore_axis_name="s")
_sc_params = pltpu.CompilerParams(needs_layout_passes=False,
                                  use_tc_tiling_on_sc=False)


@functools.partial(
    pl.kernel,
    out_type=jax.ShapeDtypeStruct((NW * DROWS * 16,), jnp.float32),
    mesh=_mesh,
    scratch_types=[
        pltpu.VMEM((EPWP,), jnp.int32),
        pltpu.VMEM((DROWS * 16,), jnp.float32),
    ],
    compiler_params=_sc_params,
)
def _sc_degree(dst_hbm, out_hbm, dstv, degloc):
    c = lax.axis_index("c")
    s = lax.axis_index("s")
    w = c * NS + s

    zero16 = jnp.zeros((16,), jnp.float32)
    one16 = jnp.ones((16,), jnp.float32)

    def _zero(r, _):
        degloc[pl.ds(r * 16, 16)] = zero16
        return 0

    lax.fori_loop(0, DROWS, _zero, 0)

    pltpu.sync_copy(dst_hbm.at[pl.ds(w * EPWP, EPWP)], dstv)

    def _hist(i, _):
        idxv = dstv[pl.ds(i * 16, 16)]
        plsc.addupdate_scatter(degloc, [idxv], one16)
        return 0

    lax.fori_loop(0, EPWP // 16, _hist, 0)

    pltpu.sync_copy(degloc, out_hbm.at[pl.ds(w * DROWS * 16, DROWS * 16)])


@functools.partial(
    pl.kernel,
    out_type=jax.ShapeDtypeStruct((NC, ACC_R, LAT), jnp.float32),
    mesh=_mesh,
    scratch_types=[
        pltpu.VMEM((NB, 128), jnp.int32),
        pltpu.VMEM((NB, 128), jnp.int32),
        pltpu.VMEM((128, LAT), jnp.float32),
        pltpu.VMEM((128, LAT), jnp.float32),
        pltpu.VMEM((128, LAT), jnp.float32),
        pltpu.VMEM((128, LAT), jnp.float32),
        pltpu.VMEM((RPT, LAT), jnp.float32),
        pltpu.VMEM_SHARED((ACC_R, LAT), jnp.float32),
        pltpu.SemaphoreType.DMA,
        pltpu.SemaphoreType.DMA,
        pltpu.SemaphoreType.DMA,
        pltpu.SemaphoreType.DMA,
    ],
    compiler_params=_sc_params,
)
def _sc_scatter(hp_hbm, src_hbm, dst_hbm, out_hbm, srcv, dstv, buf0, buf1,
                buf2, buf3, zb, acc, sem0, sem1, sem2, sem3):
    c = lax.axis_index("c")
    s = lax.axis_index("s")
    w = c * NS + s

    zero16 = jnp.zeros((16,), jnp.float32)

    def _zero(r, _):
        zb[r, pl.ds(0, 16)] = zero16
        zb[r, pl.ds(16, 16)] = zero16
        return 0

    lax.fori_loop(0, RPT, _zero, 0)
    pltpu.sync_copy(zb, acc.at[pl.ds(s * RPT, RPT)])
    plsc.subcore_barrier()

    pltpu.sync_copy(src_hbm.at[w], srcv)
    pltpu.sync_copy(dst_hbm.at[w], dstv)

    bufs = (buf0, buf1, buf2, buf3)
    sems = (sem0, sem1, sem2, sem3)
    for r in range(4):
        pltpu.async_copy(hp_hbm.at[srcv.at[r]], bufs[r], sems[r])

    def _edges(jj, _):
        j0 = 4 * jj
        for r in range(4):
            pltpu.make_async_copy(hp_hbm.at[srcv.at[j0 + r]], bufs[r],
                                  sems[r]).wait()
            pltpu.sync_copy(bufs[r], acc.at[dstv.at[j0 + r]], add=True)

            @pl.when(jj < NB // 4 - 1)
            def _():
                pltpu.async_copy(hp_hbm.at[srcv.at[j0 + 4 + r]], bufs[r],
                                 sems[r])
        return 0

    lax.fori_loop(0, NB // 4, _edges, 0)
    plsc.subcore_barrier()

    pltpu.sync_copy(acc.at[pl.ds(s * RPT, RPT)], out_hbm.at[c, pl.ds(s * RPT, RPT)])


_P = jax.lax.Precision.DEFAULT


def _tc_prep_body(degt_ref, x_ref, w1_ref, dinv_ref, hp1_ref):
    deg = jnp.sum(degt_ref[pl.ds(0, N), :], axis=1, keepdims=True) + 1.0
    dinv = lax.rsqrt(deg)
    dinv_ref[...] = dinv
    hw = lax.dot_general(x_ref[...], w1_ref[...], (((1,), (0,)), ((), ())),
                         precision=_P)
    hp1_ref[...] = hw * dinv


def _tc_prep(degt, x, w1):
    return pl.pallas_call(
        _tc_prep_body,
        out_shape=(
            jax.ShapeDtypeStruct((N, 1), jnp.float32),
            jax.ShapeDtypeStruct((N, LAT), jnp.float32),
        ),
    )(degt, x, w1)


def _tc_layer_body(agg_ref, hpp_ref, dinv_ref, w_ref, b_ref, h_ref, hpn_ref):
    agg = agg_ref[0, pl.ds(0, N), :] + agg_ref[1, pl.ds(0, N), :]
    dinv = dinv_ref[...]
    h = jnp.tanh(dinv * (agg + hpp_ref[...]) + b_ref[...])
    h_ref[...] = h
    hpn_ref[...] = dinv * lax.dot_general(h, w_ref[...], (((1,), (0,)), ((), ())),
                                          precision=_P)


def _tc_layer(aggparts, hpp, dinv, w_next, b_prev):
    return pl.pallas_call(
        _tc_layer_body,
        out_shape=(
            jax.ShapeDtypeStruct((N, LAT), jnp.float32),
            jax.ShapeDtypeStruct((N, LAT), jnp.float32),
        ),
    )(aggparts, hpp, dinv, w_next, b_prev)


def _tc_head_body(agg_ref, hp3_ref, dinv_ref, b3_ref, h1_ref, h2_ref, idx_ref,
                  y_ref, l1w_ref, l1b_ref, l2w_ref, l2b_ref,
                  logits_ref, loss_ref, acc_ref, feat_ref, h3_ref, gf_ref):
    agg = agg_ref[0, pl.ds(0, N), :] + agg_ref[1, pl.ds(0, N), :]
    h3_ref[...] = jnp.tanh(dinv_ref[...] * (agg + hp3_ref[...]) + b3_ref[...])

    for g in range(G):
        ig = idx_ref[g]
        gf_ref[pl.ds(g, 1), pl.ds(0, LAT)] = h1_ref[pl.ds(ig, 1), :]
        gf_ref[pl.ds(g, 1), pl.ds(LAT, LAT)] = h2_ref[pl.ds(ig, 1), :]
        gf_ref[pl.ds(g, 1), pl.ds(2 * LAT, LAT)] = h3_ref[pl.ds(ig, 1), :]

    hidden = lax.dot_general(gf_ref[...], l1w_ref[...], (((1,), (0,)), ((), ())),
                             precision=_P) + l1b_ref[...]
    feat_ref[...] = hidden
    hr = jnp.maximum(hidden, 0.0)
    z = lax.dot_general(hr, l2w_ref[...], (((1,), (0,)), ((), ())),
                        precision=_P) + l2b_ref[...]
    z0 = z[:, 0:1]
    z1 = z[:, 1:2]
    m = jnp.maximum(z0, z1)
    lse = m + jnp.log(jnp.exp(z0 - m) + jnp.exp(z1 - m))
    logits_ref[...] = z - lse

    ytrue = y_ref[...] == 1
    sel = jnp.where(ytrue, z1, z0) - lse
    loss_ref[...] = -jnp.sum(sel, axis=0, keepdims=True) / G
    pred1 = z1 > z0
    eq = (pred1 == ytrue).astype(jnp.float32)
    acc_ref[...] = jnp.sum(eq, axis=0, keepdims=True) / G


def _tc_head(aggparts, hp3, dinv, b3, h1, h2, idx, y2, l1w, l1b, l2w, l2b):
    return pl.pallas_call(
        _tc_head_body,
        in_specs=[
            pl.BlockSpec(memory_space=pltpu.VMEM),
            pl.BlockSpec(memory_space=pltpu.VMEM),
            pl.BlockSpec(memory_space=pltpu.VMEM),
            pl.BlockSpec(memory_space=pltpu.VMEM),
            pl.BlockSpec(memory_space=pltpu.VMEM),
            pl.BlockSpec(memory_space=pltpu.VMEM),
            pl.BlockSpec(memory_space=pltpu.SMEM),
            pl.BlockSpec(memory_space=pltpu.VMEM),
            pl.BlockSpec(memory_space=pltpu.VMEM),
            pl.BlockSpec(memory_space=pltpu.VMEM),
            pl.BlockSpec(memory_space=pltpu.VMEM),
            pl.BlockSpec(memory_space=pltpu.VMEM),
        ],
        out_shape=(
            jax.ShapeDtypeStruct((G, 2), jnp.float32),
            jax.ShapeDtypeStruct((1, 1), jnp.float32),
            jax.ShapeDtypeStruct((1, 1), jnp.float32),
            jax.ShapeDtypeStruct((G, HID), jnp.float32),
        ),
        scratch_shapes=[
            pltpu.VMEM((N, LAT), jnp.float32),
            pltpu.VMEM((G, 3 * LAT), jnp.float32),
        ],
    )(aggparts, hp3, dinv, b3, h1, h2, idx, y2, l1w, l1b, l2w, l2b)


def kernel(x, edge_index, batch, y, W1, b1, W2, b2, W3, b3, L1w, L1b, L2w, L2b):
    src = edge_index[0]
    dst = edge_index[1]

    pad = jnp.zeros((NW, EPWP - EPW), jnp.int32)
    srcp = jnp.concatenate([src.reshape(NW, EPW), pad], axis=1)
    srcp = srcp.reshape(NW, NB, 128)
    dstp = jnp.concatenate([dst.reshape(NW, EPW), pad + N], axis=1)
    dst2 = dstp.reshape(NW * EPWP)
    dstp = dstp.reshape(NW, NB, 128)

    mask = jnp.concatenate([jnp.ones((1,), jnp.bool_), batch[1:] != batch[:-1]])
    idx = jnp.nonzero(mask, size=G, fill_value=0)[0].astype(jnp.int32)

    degparts = _sc_degree(dst2)
    degt = jnp.transpose(degparts.reshape(NW, DROWS * 16), (1, 0))

    dinv, hp1 = _tc_prep(degt, x, W1)

    agg1 = _sc_scatter(hp1, srcp, dstp)
    h1, hp2 = _tc_layer(agg1, hp1, dinv, W2, b1.reshape(1, LAT))
    agg2 = _sc_scatter(hp2, srcp, dstp)
    h2, hp3 = _tc_layer(agg2, hp2, dinv, W3, b2.reshape(1, LAT))
    agg3 = _sc_scatter(hp3, srcp, dstp)

    logits, loss, acc, feature = _tc_head(
        agg3, hp3, dinv, b3.reshape(1, LAT), h1, h2, idx,
        y.reshape(G, 1).astype(jnp.int32), L1w, L1b.reshape(1, HID),
        L2w, L2b.reshape(1, 2))

    return (logits, loss.reshape(()), acc.reshape(()), feature)


# speedup vs baseline: 1.1294x; 1.0271x over previous
import functools

import jax
import jax.numpy as jnp
from jax import lax
from jax.experimental import pallas as pl
from jax.experimental.pallas import tpu as pltpu
from jax.experimental.pallas import tpu_sc as plsc

N = 10000
E = 320000
G = 64
D_IN = 128
LAT = 32
HID = 512

NC = 2
NS = 16
NW = NC * NS
EPW = E // NW
NB = 80
EPWP = NB * 128
DROWS = 640
ACC_R = 10112
RPT = ACC_R // NS

_mesh = plsc.VectorSubcoreMesh(core_axis_name="c", subcore_axis_name="s")
_sc_params = pltpu.CompilerParams(needs_layout_passes=False,
                                  use_tc_tiling_on_sc=False)


@functools.partial(
    pl.kernel,
    out_type=jax.ShapeDtypeStruct((NW * DROWS * 16,), jnp.float32),
    mesh=_mesh,
    scratch_types=[
        pltpu.VMEM((EPWP,), jnp.int32),
        pltpu.VMEM((DROWS * 16,), jnp.float32),
    ],
    compiler_params=_sc_params,
)
def _sc_degree(dst_hbm, out_hbm, dstv, degloc):
    c = lax.axis_index("c")
    s = lax.axis_index("s")
    w = c * NS + s

    zero16 = jnp.zeros((16,), jnp.float32)
    one16 = jnp.ones((16,), jnp.float32)

    def _zero(r, _):
        degloc[pl.ds(r * 16, 16)] = zero16
        return 0

    lax.fori_loop(0, DROWS, _zero, 0)

    pltpu.sync_copy(dst_hbm.at[pl.ds(w * EPWP, EPWP)], dstv)

    def _hist(i, _):
        idxv = dstv[pl.ds(i * 16, 16)]
        plsc.addupdate_scatter(degloc, [idxv], one16)
        return 0

    lax.fori_loop(0, EPWP // 16, _hist, 0)

    pltpu.sync_copy(degloc, out_hbm.at[pl.ds(w * DROWS * 16, DROWS * 16)])


@functools.partial(
    pl.kernel,
    out_type=jax.ShapeDtypeStruct((NC, ACC_R, LAT), jnp.float32),
    mesh=_mesh,
    scratch_types=[
        pltpu.VMEM((NB, 128), jnp.int32),
        pltpu.VMEM((NB, 128), jnp.int32),
        pltpu.VMEM((128, LAT), jnp.float32),
        pltpu.VMEM((128, LAT), jnp.float32),
        pltpu.VMEM((128, LAT), jnp.float32),
        pltpu.VMEM((128, LAT), jnp.float32),
        pltpu.VMEM((128, LAT), jnp.float32),
        pltpu.VMEM((128, LAT), jnp.float32),
        pltpu.VMEM((128, LAT), jnp.float32),
        pltpu.VMEM((128, LAT), jnp.float32),
        pltpu.VMEM((RPT, LAT), jnp.float32),
        pltpu.VMEM_SHARED((ACC_R, LAT), jnp.float32),
        pltpu.SemaphoreType.DMA,
        pltpu.SemaphoreType.DMA,
        pltpu.SemaphoreType.DMA,
        pltpu.SemaphoreType.DMA,
        pltpu.SemaphoreType.DMA,
        pltpu.SemaphoreType.DMA,
        pltpu.SemaphoreType.DMA,
        pltpu.SemaphoreType.DMA,
    ],
    compiler_params=_sc_params,
)
def _sc_scatter(hp_hbm, src_hbm, dst_hbm, out_hbm, srcv, dstv, buf0, buf1,
                buf2, buf3, buf4, buf5, buf6, buf7, zb, acc,
                sem0, sem1, sem2, sem3, sem4, sem5, sem6, sem7):
    c = lax.axis_index("c")
    s = lax.axis_index("s")
    w = c * NS + s

    zero16 = jnp.zeros((16,), jnp.float32)

    def _zero(r, _):
        zb[r, pl.ds(0, 16)] = zero16
        zb[r, pl.ds(16, 16)] = zero16
        return 0

    lax.fori_loop(0, RPT, _zero, 0)
    pltpu.sync_copy(zb, acc.at[pl.ds(s * RPT, RPT)])
    plsc.subcore_barrier()

    pltpu.sync_copy(src_hbm.at[w], srcv)
    pltpu.sync_copy(dst_hbm.at[w], dstv)

    bufs = (buf0, buf1, buf2, buf3, buf4, buf5, buf6, buf7)
    sems = (sem0, sem1, sem2, sem3, sem4, sem5, sem6, sem7)
    for r in range(8):
        pltpu.async_copy(hp_hbm.at[srcv.at[r]], bufs[r], sems[r])

    def _edges(jj, _):
        j0 = 8 * jj
        for r in range(8):
            pltpu.make_async_copy(hp_hbm.at[srcv.at[j0 + r]], bufs[r],
                                  sems[r]).wait()
            pltpu.sync_copy(bufs[r], acc.at[dstv.at[j0 + r]], add=True)

            @pl.when(jj < NB // 8 - 1)
            def _():
                pltpu.async_copy(hp_hbm.at[srcv.at[j0 + 8 + r]], bufs[r],
                                 sems[r])
        return 0

    lax.fori_loop(0, NB // 8, _edges, 0)
    plsc.subcore_barrier()

    pltpu.sync_copy(acc.at[pl.ds(s * RPT, RPT)], out_hbm.at[c, pl.ds(s * RPT, RPT)])


_P = jax.lax.Precision.DEFAULT


def _tc_prep_body(degt_ref, x_ref, w1_ref, dinv_ref, hp1_ref):
    deg = jnp.sum(degt_ref[pl.ds(0, N), :], axis=1, keepdims=True) + 1.0
    dinv = lax.rsqrt(deg)
    dinv_ref[...] = dinv
    hw = lax.dot_general(x_ref[...], w1_ref[...], (((1,), (0,)), ((), ())),
                         precision=_P)
    hp1_ref[...] = hw * dinv


def _tc_prep(degt, x, w1):
    return pl.pallas_call(
        _tc_prep_body,
        out_shape=(
            jax.ShapeDtypeStruct((N, 1), jnp.float32),
            jax.ShapeDtypeStruct((N, LAT), jnp.float32),
        ),
    )(degt, x, w1)


def _tc_layer_body(agg_ref, hpp_ref, dinv_ref, w_ref, b_ref, h_ref, hpn_ref):
    agg = agg_ref[0, pl.ds(0, N), :] + agg_ref[1, pl.ds(0, N), :]
    dinv = dinv_ref[...]
    h = jnp.tanh(dinv * (agg + hpp_ref[...]) + b_ref[...])
    h_ref[...] = h
    hpn_ref[...] = dinv * lax.dot_general(h, w_ref[...], (((1,), (0,)), ((), ())),
                                          precision=_P)


def _tc_layer(aggparts, hpp, dinv, w_next, b_prev):
    return pl.pallas_call(
        _tc_layer_body,
        out_shape=(
            jax.ShapeDtypeStruct((N, LAT), jnp.float32),
            jax.ShapeDtypeStruct((N, LAT), jnp.float32),
        ),
    )(aggparts, hpp, dinv, w_next, b_prev)


def _tc_head_body(agg_ref, hp3_ref, dinv_ref, b3_ref, h1_ref, h2_ref, idx_ref,
                  y_ref, l1w_ref, l1b_ref, l2w_ref, l2b_ref,
                  logits_ref, loss_ref, acc_ref, feat_ref, h3_ref, gf_ref):
    agg = agg_ref[0, pl.ds(0, N), :] + agg_ref[1, pl.ds(0, N), :]
    h3_ref[...] = jnp.tanh(dinv_ref[...] * (agg + hp3_ref[...]) + b3_ref[...])

    for g in range(G):
        ig = idx_ref[g]
        gf_ref[pl.ds(g, 1), pl.ds(0, LAT)] = h1_ref[pl.ds(ig, 1), :]
        gf_ref[pl.ds(g, 1), pl.ds(LAT, LAT)] = h2_ref[pl.ds(ig, 1), :]
        gf_ref[pl.ds(g, 1), pl.ds(2 * LAT, LAT)] = h3_ref[pl.ds(ig, 1), :]

    hidden = lax.dot_general(gf_ref[...], l1w_ref[...], (((1,), (0,)), ((), ())),
                             precision=_P) + l1b_ref[...]
    feat_ref[...] = hidden
    hr = jnp.maximum(hidden, 0.0)
    z = lax.dot_general(hr, l2w_ref[...], (((1,), (0,)), ((), ())),
                        precision=_P) + l2b_ref[...]
    z0 = z[:, 0:1]
    z1 = z[:, 1:2]
    m = jnp.maximum(z0, z1)
    lse = m + jnp.log(jnp.exp(z0 - m) + jnp.exp(z1 - m))
    logits_ref[...] = z - lse

    ytrue = y_ref[...] == 1
    sel = jnp.where(ytrue, z1, z0) - lse
    loss_ref[...] = -jnp.sum(sel, axis=0, keepdims=True) / G
    pred1 = z1 > z0
    eq = (pred1 == ytrue).astype(jnp.float32)
    acc_ref[...] = jnp.sum(eq, axis=0, keepdims=True) / G


def _tc_head(aggparts, hp3, dinv, b3, h1, h2, idx, y2, l1w, l1b, l2w, l2b):
    return pl.pallas_call(
        _tc_head_body,
        in_specs=[
            pl.BlockSpec(memory_space=pltpu.VMEM),
            pl.BlockSpec(memory_space=pltpu.VMEM),
            pl.BlockSpec(memory_space=pltpu.VMEM),
            pl.BlockSpec(memory_space=pltpu.VMEM),
            pl.BlockSpec(memory_space=pltpu.VMEM),
            pl.BlockSpec(memory_space=pltpu.VMEM),
            pl.BlockSpec(memory_space=pltpu.SMEM),
            pl.BlockSpec(memory_space=pltpu.VMEM),
            pl.BlockSpec(memory_space=pltpu.VMEM),
            pl.BlockSpec(memory_space=pltpu.VMEM),
            pl.BlockSpec(memory_space=pltpu.VMEM),
            pl.BlockSpec(memory_space=pltpu.VMEM),
        ],
        out_shape=(
            jax.ShapeDtypeStruct((G, 2), jnp.float32),
            jax.ShapeDtypeStruct((1, 1), jnp.float32),
            jax.ShapeDtypeStruct((1, 1), jnp.float32),
            jax.ShapeDtypeStruct((G, HID), jnp.float32),
        ),
        scratch_shapes=[
            pltpu.VMEM((N, LAT), jnp.float32),
            pltpu.VMEM((G, 3 * LAT), jnp.float32),
        ],
    )(aggparts, hp3, dinv, b3, h1, h2, idx, y2, l1w, l1b, l2w, l2b)


def kernel(x, edge_index, batch, y, W1, b1, W2, b2, W3, b3, L1w, L1b, L2w, L2b):
    src = edge_index[0]
    dst = edge_index[1]

    pad = jnp.zeros((NW, EPWP - EPW), jnp.int32)
    srcp = jnp.concatenate([src.reshape(NW, EPW), pad], axis=1)
    srcp = srcp.reshape(NW, NB, 128)
    dstp = jnp.concatenate([dst.reshape(NW, EPW), pad + N], axis=1)
    dst2 = dstp.reshape(NW * EPWP)
    dstp = dstp.reshape(NW, NB, 128)

    mask = jnp.concatenate([jnp.ones((1,), jnp.bool_), batch[1:] != batch[:-1]])
    idx = jnp.nonzero(mask, size=G, fill_value=0)[0].astype(jnp.int32)

    degparts = _sc_degree(dst2)
    degt = jnp.transpose(degparts.reshape(NW, DROWS * 16), (1, 0))

    dinv, hp1 = _tc_prep(degt, x, W1)

    agg1 = _sc_scatter(hp1, srcp, dstp)
    h1, hp2 = _tc_layer(agg1, hp1, dinv, W2, b1.reshape(1, LAT))
    agg2 = _sc_scatter(hp2, srcp, dstp)
    h2, hp3 = _tc_layer(agg2, hp2, dinv, W3, b2.reshape(1, LAT))
    agg3 = _sc_scatter(hp3, srcp, dstp)

    logits, loss, acc, feature = _tc_head(
        agg3, hp3, dinv, b3.reshape(1, LAT), h1, h2, idx,
        y.reshape(G, 1).astype(jnp.int32), L1w, L1b.reshape(1, HID),
        L2w, L2b.reshape(1, 2))

    return (logits, loss.reshape(()), acc.reshape(()), feature)
